# baseline (speedup 1.0000x reference)
import functools
import jax
import jax.numpy as jnp
from jax import lax
from jax.experimental import pallas as pl
from jax.experimental.pallas import tpu as pltpu, tpu_sc as plsc

_SCALE_BOUND = 0.11
_INV_SQRT2 = 0.7071067811865476

_N = 192 * 256 * 256
_NC, _NS, _L = 2, 16, 16
_NW = _NC * _NS
_PER_W = _N // _NW
_T = 8192
_TILES = _PER_W // _T
_VECS = _T // _L

_P = 0.3275911
_A1 = 0.254829592
_A2 = -0.284496736
_A3 = 1.421413741
_A4 = -1.453152027
_A5 = 1.061405429


def _erf(z):
    az = jnp.abs(z)
    t = 1.0 / (1.0 + _P * az)
    poly = ((((_A5 * t + _A4) * t + _A3) * t + _A2) * t + _A1) * t
    r = 1.0 - poly * jnp.exp(-az * az)
    return jnp.where(z < 0.0, -r, r)


def _sc_body(x_hbm, c_hbm, lik_hbm, xout_hbm, xs, ms, ss, ls):
    wid = lax.axis_index("s") * _NC + lax.axis_index("c")
    base = wid * _PER_W

    def tile_body(tt, _):
        start = base + tt * _T
        pltpu.sync_copy(x_hbm.at[pl.ds(start, _T)], xs)
        pltpu.sync_copy(c_hbm.at[pl.ds(start, _T)], ms)
        pltpu.sync_copy(c_hbm.at[pl.ds(_N + start, _T)], ss)

        def vec_body(i, _):
            off = i * _L
            x = xs[pl.ds(off, _L)]
            m = ms[pl.ds(off, _L)]
            s = jnp.maximum(ss[pl.ds(off, _L)], _SCALE_BOUND)
            v = jnp.abs(x - m)
            c = _INV_SQRT2 / s
            lik = 0.5 * (_erf((0.5 - v) * c) - _erf((-0.5 - v) * c))
            ls[pl.ds(off, _L)] = lik
            return _

        lax.fori_loop(0, _VECS, vec_body, None, unroll=4)
        pltpu.sync_copy(ls, lik_hbm.at[pl.ds(start, _T)])
        pltpu.sync_copy(xs, xout_hbm.at[pl.ds(start, _T)])
        return _

    lax.fori_loop(0, _TILES, tile_body, None)


def kernel(input, condition):
    xf = input.reshape(_N)
    cf = condition.reshape(2 * _N)
    mesh = plsc.VectorSubcoreMesh(
        core_axis_name="c", subcore_axis_name="s",
        num_cores=_NC, num_subcores=_NS,
    )
    lik, xout = pl.kernel(
        _sc_body,
        out_type=[
            jax.ShapeDtypeStruct((_N,), jnp.float32),
            jax.ShapeDtypeStruct((_N,), jnp.float32),
        ],
        mesh=mesh,
        scratch_types=[
            pltpu.VMEM((_T,), jnp.float32),
            pltpu.VMEM((_T,), jnp.float32),
            pltpu.VMEM((_T,), jnp.float32),
            pltpu.VMEM((_T,), jnp.float32),
        ],
    )(xf, cf)
    return (xout.reshape(input.shape), lik.reshape(input.shape))


# speedup vs baseline: 17.0239x; 17.0239x over previous
import jax
import jax.numpy as jnp
from jax import lax
from jax.experimental import pallas as pl
from jax.experimental.pallas import tpu as pltpu

_SCALE_BOUND = 0.11
_LANES = 256
_ROWS = 192 * 256 * 256 // _LANES
_BLOCK_ROWS = 4096
_INV_SQRT2 = 0.7071067811865476


def _body(x_ref, m_ref, s_ref, o_ref, xo_ref):
    x = x_ref[...]
    m = m_ref[0]
    s = jnp.maximum(s_ref[0], _SCALE_BOUND)
    v = jnp.abs(x - m)
    c = _INV_SQRT2 / s
    upper = lax.erf((0.5 - v) * c)
    lower = lax.erf((-0.5 - v) * c)
    o_ref[...] = 0.5 * (upper - lower)
    xo_ref[...] = x


def kernel(input, condition):
    x2 = input.reshape(_ROWS, _LANES)
    c3 = condition.reshape(2, _ROWS, _LANES)
    grid = _ROWS // _BLOCK_ROWS
    lik = pl.pallas_call(
        _body,
        grid=(grid,),
        in_specs=[
            pl.BlockSpec((_BLOCK_ROWS, _LANES), lambda i: (i, 0)),
            pl.BlockSpec((1, _BLOCK_ROWS, _LANES), lambda i: (0, i, 0)),
            pl.BlockSpec((1, _BLOCK_ROWS, _LANES), lambda i: (1, i, 0)),
        ],
        out_specs=[
            pl.BlockSpec((_BLOCK_ROWS, _LANES), lambda i: (i, 0)),
            pl.BlockSpec((_BLOCK_ROWS, _LANES), lambda i: (i, 0)),
        ],
        out_shape=[
            jax.ShapeDtypeStruct((_ROWS, _LANES), jnp.float32),
            jax.ShapeDtypeStruct((_ROWS, _LANES), jnp.float32),
        ],
        compiler_params=pltpu.CompilerParams(
            dimension_semantics=("parallel",),
        ),
    )(x2, c3, c3)
    lik, xout = lik
    return (xout.reshape(input.shape), lik.reshape(input.shape))
